# Initial kernel scaffold; baseline (speedup 1.0000x reference)
import jax
import jax.numpy as jnp
from jax.experimental import pallas as pl


def kernel(x, edge_index, W1, b1, gamma, beta, W2, b2):
    raise NotImplementedError("write your pallas kernel here")


# speedup vs baseline: 10.2563x; 10.2563x over previous
import functools

import jax
import jax.numpy as jnp
from jax import lax
from jax.experimental import pallas as pl
from jax.experimental.pallas import tpu as pltpu
from jax.experimental.pallas import tpu_sc as plsc

N = 10000
NPAD = 10240
E = 320000
D = 128
DH = D // 2
NC, NS = 2, 16
NW = NC * NS
CHUNK = 128
G = 80
G2 = 160
EPAD = NW * G * CHUNK
RPT = NPAD // NS
DEG_W = 8

_MESH = plsc.VectorSubcoreMesh(core_axis_name="c", subcore_axis_name="s")


def _deg_body(dstb, ones_hbm, zeros_hbm, out, acc_sh, dst_v, ones_v):
    c = lax.axis_index("c")
    s = lax.axis_index("s")
    w = s * NC + c
    pltpu.sync_copy(zeros_hbm.at[pl.ds(s * RPT, RPT)],
                    acc_sh.at[pl.ds(s * RPT, RPT)])
    pltpu.sync_copy(dstb.at[w], dst_v)
    pltpu.sync_copy(ones_hbm, ones_v)
    plsc.subcore_barrier()

    def body(g, carry):
        pltpu.sync_copy(ones_v, acc_sh.at[dst_v.at[g]], add=True)
        return carry

    lax.fori_loop(0, G, body, 0)
    plsc.subcore_barrier()
    pltpu.sync_copy(acc_sh.at[pl.ds(s * RPT, RPT)],
                    out.at[c].at[pl.ds(s * RPT, RPT)])


_deg_call = pl.kernel(
    _deg_body,
    out_type=jax.ShapeDtypeStruct((NC, NPAD, DEG_W), jnp.float32),
    mesh=_MESH,
    compiler_params=pltpu.CompilerParams(use_tc_tiling_on_sc=False),
    scratch_types=[
        pltpu.VMEM_SHARED((NPAD, DEG_W), jnp.float32),
        pltpu.VMEM((G, CHUNK), jnp.int32),
        pltpu.VMEM((CHUNK, DEG_W), jnp.float32),
    ],
)


def _segsum_body(y_hbm, srcb, dstb, zeros_hbm, out,
                 acc_sh, src_v, dst_v, rows0, rows1, sem0, sem1):
    c = lax.axis_index("c")
    s = lax.axis_index("s")
    w = c * NS + s
    pltpu.sync_copy(zeros_hbm.at[pl.ds(s * RPT, RPT)],
                    acc_sh.at[pl.ds(s * RPT, RPT)])
    pltpu.sync_copy(srcb.at[w], src_v)
    pltpu.sync_copy(dstb.at[s], dst_v)
    plsc.subcore_barrier()

    pltpu.async_copy(y_hbm.at[src_v.at[0]], rows0, sem0)

    def body(p, carry):
        g0 = 2 * p
        g1 = g0 + 1
        pltpu.make_async_copy(y_hbm.at[src_v.at[g0]], rows0, sem0).wait()
        pltpu.async_copy(y_hbm.at[src_v.at[g1]], rows1, sem1)
        pltpu.sync_copy(rows0, acc_sh.at[dst_v.at[g0]], add=True)
        pltpu.make_async_copy(y_hbm.at[src_v.at[g1]], rows1, sem1).wait()

        @pl.when(g1 + 1 < G2)
        def _():
            pltpu.async_copy(y_hbm.at[src_v.at[g0 + 2]], rows0, sem0)

        pltpu.sync_copy(rows1, acc_sh.at[dst_v.at[g1]], add=True)
        return carry

    lax.fori_loop(0, G2 // 2, body, 0)
    plsc.subcore_barrier()
    pltpu.sync_copy(acc_sh.at[pl.ds(s * RPT, RPT)],
                    out.at[c].at[pl.ds(s * RPT, RPT)])


_segsum_call = pl.kernel(
    _segsum_body,
    out_type=jax.ShapeDtypeStruct((NC, NPAD, DH), jnp.float32),
    mesh=_MESH,
    compiler_params=pltpu.CompilerParams(use_tc_tiling_on_sc=False),
    scratch_types=[
        pltpu.VMEM_SHARED((NPAD, DH), jnp.float32),
        pltpu.VMEM((G2, CHUNK), jnp.int32),
        pltpu.VMEM((G2, CHUNK), jnp.int32),
        pltpu.VMEM((CHUNK, DH), jnp.float32),
        pltpu.VMEM((CHUNK, DH), jnp.float32),
        pltpu.SemaphoreType.DMA,
        pltpu.SemaphoreType.DMA,
    ],
)


def _tc1_body(degp, x_ref, w1_ref, y1_ref, dinv_ref):
    deg = degp[0] + degp[1]
    deg0 = deg[:, 0:1]
    dinv = lax.rsqrt(jnp.maximum(deg0, 1.0))
    dinv_ref[...] = dinv
    y = jnp.dot(x_ref[...] * dinv, w1_ref[...],
                preferred_element_type=jnp.float32)
    y1_ref[0] = y[:, :DH]
    y1_ref[1] = y[:, DH:]


def _tc2_body(z1p, dinv_ref, b1_ref, gamma_ref, beta_ref, w2_ref, y2_ref):
    z = jnp.concatenate([z1p[0], z1p[1]], axis=1)
    dinv = dinv_ref[...]
    h = z * dinv + b1_ref[...]
    rows = lax.broadcasted_iota(jnp.int32, (NPAD, 1), 0)
    mask = (rows < N).astype(jnp.float32)
    hm = h * mask
    mean = jnp.sum(hm, axis=0, keepdims=True) * (1.0 / N)
    var = jnp.sum(hm * hm, axis=0, keepdims=True) * (1.0 / N) - mean * mean
    hn = gamma_ref[...] * (h - mean) * lax.rsqrt(var + 1e-5) + beta_ref[...]
    hr = jnp.maximum(hn, 0.0)
    y = jnp.dot(hr * dinv * mask, w2_ref[...],
                preferred_element_type=jnp.float32)
    y2_ref[0] = y[:, :DH]
    y2_ref[1] = y[:, DH:]


def _tc3_body(z2p, dinv_ref, b2_ref, out_ref):
    z = jnp.concatenate([z2p[0], z2p[1]], axis=1)
    out_ref[...] = z * dinv_ref[...] + b2_ref[...]


_tc1_call = pl.pallas_call(
    _tc1_body,
    out_shape=(jax.ShapeDtypeStruct((NC, NPAD, DH), jnp.float32),
               jax.ShapeDtypeStruct((NPAD, 1), jnp.float32)),
)

_tc2_call = pl.pallas_call(
    _tc2_body,
    out_shape=jax.ShapeDtypeStruct((NC, NPAD, DH), jnp.float32),
)

_tc3_call = pl.pallas_call(
    _tc3_body,
    out_shape=jax.ShapeDtypeStruct((NPAD, D), jnp.float32),
)


@jax.jit
def kernel(x, edge_index, W1, b1, gamma, beta, W2, b2):
    src = edge_index[0].astype(jnp.int32)
    dst = edge_index[1].astype(jnp.int32)
    pad = jnp.full((EPAD - E,), N, jnp.int32)
    src_p = jnp.concatenate([src, pad])
    dst_p = jnp.concatenate([dst, pad])
    dstb32 = dst_p.reshape(NW, G, CHUNK)
    srcb = jnp.stack([src_p, src_p + NPAD]).reshape(NC * NS, G2, CHUNK)
    dstb = dst_p.reshape(NS, G2, CHUNK)
    x_p = jnp.concatenate(
        [x, jnp.zeros((NPAD - N, D), jnp.float32)], axis=0)

    ones8 = jnp.ones((CHUNK, DEG_W), jnp.float32)
    zeros8 = jnp.zeros((NPAD, DEG_W), jnp.float32)
    zeros64 = jnp.zeros((NPAD, DH), jnp.float32)

    b1r = b1.reshape(1, D)
    b2r = b2.reshape(1, D)
    gammar = gamma.reshape(1, D)
    betar = beta.reshape(1, D)

    degp = _deg_call(dstb32, ones8, zeros8)
    y1, dinv = _tc1_call(degp, x_p, W1)
    z1p = _segsum_call(y1.reshape(NC * NPAD, DH), srcb, dstb, zeros64)
    y2 = _tc2_call(z1p, dinv, b1r, gammar, betar, W2)
    z2p = _segsum_call(y2.reshape(NC * NPAD, DH), srcb, dstb, zeros64)
    out_p = _tc3_call(z2p, dinv, b2r)
    return out_p[:N]


# speedup vs baseline: 11.5581x; 1.1269x over previous
import functools

import jax
import jax.numpy as jnp
from jax import lax
from jax.experimental import pallas as pl
from jax.experimental.pallas import tpu as pltpu
from jax.experimental.pallas import tpu_sc as plsc

N = 10000
NPAD = 10240
E = 320000
D = 128
DH = D // 2
NC, NS = 2, 16
NW = NC * NS
CHUNK = 128
G = 80
G2 = 160
EPAD = NW * G * CHUNK
RPT = NPAD // NS
DEG_W = 8

_MESH = plsc.VectorSubcoreMesh(core_axis_name="c", subcore_axis_name="s")


def _deg_body(dstb, ones_hbm, zeros_hbm, out, acc_sh, dst_v, ones_v, dsem):
    c = lax.axis_index("c")
    s = lax.axis_index("s")
    w = s * NC + c
    pltpu.sync_copy(zeros_hbm.at[pl.ds(s * RPT, RPT)],
                    acc_sh.at[pl.ds(s * RPT, RPT)])
    pltpu.sync_copy(dstb.at[w], dst_v)
    pltpu.sync_copy(ones_hbm, ones_v)
    plsc.subcore_barrier()

    def body(q, carry):
        g = 8 * q
        for k in range(8):
            pltpu.async_copy(ones_v, acc_sh.at[dst_v.at[g + k]], dsem,
                             add=True)
        for k in range(8):
            pltpu.make_async_copy(ones_v, acc_sh.at[dst_v.at[g + k]],
                                  dsem).wait()
        return carry

    lax.fori_loop(0, G // 8, body, 0)
    plsc.subcore_barrier()
    pltpu.sync_copy(acc_sh.at[pl.ds(s * RPT, RPT)],
                    out.at[c].at[pl.ds(s * RPT, RPT)])


_deg_call = pl.kernel(
    _deg_body,
    out_type=jax.ShapeDtypeStruct((NC, NPAD, DEG_W), jnp.float32),
    mesh=_MESH,
    compiler_params=pltpu.CompilerParams(use_tc_tiling_on_sc=False),
    scratch_types=[
        pltpu.VMEM_SHARED((NPAD, DEG_W), jnp.float32),
        pltpu.VMEM((G, CHUNK), jnp.int32),
        pltpu.VMEM((CHUNK, DEG_W), jnp.float32),
        pltpu.SemaphoreType.DMA,
    ],
)


_NBUF = 4


def _segsum_body(y_hbm, srcb, dstb, zeros_hbm, out,
                 acc_sh, src_v, dst_v, rows0, rows1, rows2, rows3,
                 gs0, gs1, gs2, gs3, ss0, ss1, ss2, ss3):
    rows = (rows0, rows1, rows2, rows3)
    gsem = (gs0, gs1, gs2, gs3)
    ssem = (ss0, ss1, ss2, ss3)
    c = lax.axis_index("c")
    s = lax.axis_index("s")
    w = c * NS + s
    pltpu.sync_copy(zeros_hbm.at[pl.ds(s * RPT, RPT)],
                    acc_sh.at[pl.ds(s * RPT, RPT)])
    pltpu.sync_copy(srcb.at[w], src_v)
    pltpu.sync_copy(dstb.at[s], dst_v)
    plsc.subcore_barrier()

    for b in range(_NBUF):
        pltpu.async_copy(y_hbm.at[src_v.at[b]], rows[b], gsem[b])

    def body(p, carry):
        g = _NBUF * p
        for b in range(_NBUF):
            pltpu.make_async_copy(y_hbm.at[src_v.at[g + b]], rows[b],
                                  gsem[b]).wait()
            pltpu.async_copy(rows[b], acc_sh.at[dst_v.at[g + b]], ssem[b],
                             add=True)
        for b in range(_NBUF):
            @pl.when(g + b + _NBUF < G2)
            def _(b=b, g=g):
                pltpu.make_async_copy(rows[b], acc_sh.at[dst_v.at[g + b]],
                                      ssem[b]).wait()
                pltpu.async_copy(y_hbm.at[src_v.at[g + b + _NBUF]], rows[b],
                                 gsem[b])
        return carry

    lax.fori_loop(0, G2 // _NBUF, body, 0)
    for b in range(_NBUF):
        pltpu.make_async_copy(rows[b], acc_sh.at[dst_v.at[G2 - _NBUF + b]],
                              ssem[b]).wait()
    plsc.subcore_barrier()
    pltpu.sync_copy(acc_sh.at[pl.ds(s * RPT, RPT)],
                    out.at[c].at[pl.ds(s * RPT, RPT)])


_segsum_call = pl.kernel(
    _segsum_body,
    out_type=jax.ShapeDtypeStruct((NC, NPAD, DH), jnp.float32),
    mesh=_MESH,
    compiler_params=pltpu.CompilerParams(use_tc_tiling_on_sc=False),
    scratch_types=[
        pltpu.VMEM_SHARED((NPAD, DH), jnp.float32),
        pltpu.VMEM((G2, CHUNK), jnp.int32),
        pltpu.VMEM((G2, CHUNK), jnp.int32),
        pltpu.VMEM((CHUNK, DH), jnp.float32),
        pltpu.VMEM((CHUNK, DH), jnp.float32),
        pltpu.VMEM((CHUNK, DH), jnp.float32),
        pltpu.VMEM((CHUNK, DH), jnp.float32),
        pltpu.SemaphoreType.DMA,
        pltpu.SemaphoreType.DMA,
        pltpu.SemaphoreType.DMA,
        pltpu.SemaphoreType.DMA,
        pltpu.SemaphoreType.DMA,
        pltpu.SemaphoreType.DMA,
        pltpu.SemaphoreType.DMA,
        pltpu.SemaphoreType.DMA,
    ],
)


def _tc1_body(degp, x_ref, w1_ref, y1_ref, dinv_ref):
    deg = degp[0] + degp[1]
    deg0 = deg[:, 0:1]
    dinv = lax.rsqrt(jnp.maximum(deg0, 1.0))
    dinv_ref[...] = dinv
    y = jnp.dot(x_ref[...] * dinv, w1_ref[...],
                preferred_element_type=jnp.float32)
    y1_ref[0] = y[:, :DH]
    y1_ref[1] = y[:, DH:]


def _tc2_body(z1p, dinv_ref, b1_ref, gamma_ref, beta_ref, w2_ref, y2_ref):
    z = jnp.concatenate([z1p[0], z1p[1]], axis=1)
    dinv = dinv_ref[...]
    h = z * dinv + b1_ref[...]
    rows = lax.broadcasted_iota(jnp.int32, (NPAD, 1), 0)
    mask = (rows < N).astype(jnp.float32)
    hm = h * mask
    mean = jnp.sum(hm, axis=0, keepdims=True) * (1.0 / N)
    var = jnp.sum(hm * hm, axis=0, keepdims=True) * (1.0 / N) - mean * mean
    hn = gamma_ref[...] * (h - mean) * lax.rsqrt(var + 1e-5) + beta_ref[...]
    hr = jnp.maximum(hn, 0.0)
    y = jnp.dot(hr * dinv * mask, w2_ref[...],
                preferred_element_type=jnp.float32)
    y2_ref[0] = y[:, :DH]
    y2_ref[1] = y[:, DH:]


def _tc3_body(z2p, dinv_ref, b2_ref, out_ref):
    z = jnp.concatenate([z2p[0], z2p[1]], axis=1)
    out_ref[...] = z * dinv_ref[...] + b2_ref[...]


_tc1_call = pl.pallas_call(
    _tc1_body,
    out_shape=(jax.ShapeDtypeStruct((NC, NPAD, DH), jnp.float32),
               jax.ShapeDtypeStruct((NPAD, 1), jnp.float32)),
)

_tc2_call = pl.pallas_call(
    _tc2_body,
    out_shape=jax.ShapeDtypeStruct((NC, NPAD, DH), jnp.float32),
)

_tc3_call = pl.pallas_call(
    _tc3_body,
    out_shape=jax.ShapeDtypeStruct((NPAD, D), jnp.float32),
)


@jax.jit
def kernel(x, edge_index, W1, b1, gamma, beta, W2, b2):
    src = edge_index[0].astype(jnp.int32)
    dst = edge_index[1].astype(jnp.int32)
    pad = jnp.full((EPAD - E,), N, jnp.int32)
    src_p = jnp.concatenate([src, pad])
    dst_p = jnp.concatenate([dst, pad])
    dstb32 = dst_p.reshape(NW, G, CHUNK)
    srcb = jnp.stack([src_p, src_p + NPAD]).reshape(NC * NS, G2, CHUNK)
    dstb = dst_p.reshape(NS, G2, CHUNK)
    x_p = jnp.concatenate(
        [x, jnp.zeros((NPAD - N, D), jnp.float32)], axis=0)

    ones8 = jnp.ones((CHUNK, DEG_W), jnp.float32)
    zeros8 = jnp.zeros((NPAD, DEG_W), jnp.float32)
    zeros64 = jnp.zeros((NPAD, DH), jnp.float32)

    b1r = b1.reshape(1, D)
    b2r = b2.reshape(1, D)
    gammar = gamma.reshape(1, D)
    betar = beta.reshape(1, D)

    degp = _deg_call(dstb32, ones8, zeros8)
    y1, dinv = _tc1_call(degp, x_p, W1)
    z1p = _segsum_call(y1.reshape(NC * NPAD, DH), srcb, dstb, zeros64)
    y2 = _tc2_call(z1p, dinv, b1r, gammar, betar, W2)
    z2p = _segsum_call(y2.reshape(NC * NPAD, DH), srcb, dstb, zeros64)
    out_p = _tc3_call(z2p, dinv, b2r)
    return out_p[:N]
